# baseline (speedup 1.0000x reference)
import math

import jax
import jax.numpy as jnp
from jax import lax
from jax.experimental import pallas as pl
from jax.experimental.pallas import tpu as pltpu
from jax.experimental.pallas import tpu_sc as plsc

BATCH = 4096
HIST = 200
D = 32
L = 16
GLOBAL_MEAN = 3.5

NC, NS = 2, 16
NW = NC * NS
BPW = BATCH // NW
NCH = BPW // L
NB = 8
YJN = 16
YB = 8
WIDTHS = (32, 64, 96, 128)


def _svdpp_kernel(user_idx_hbm, item_idx_hbm, iu_hbm, ueT_hbm, ieT_hbm,
                  ubT_hbm, ibT_hbm, yjT_hbm, out_hbm,
                  uidx_v, iidx_v, iu1_v, ubuf, ibuf, yjbuf, ub_v, ib_v,
                  tmp_v, gath_v, out_v, impl_sh,
                  semu0, semu1, semu2, semu3, semu4, semu5, semu6, semu7,
                  semy0, semy1, semy2, semy3, semy4, semy5, semy6, semy7,
                  semb, semi, semi2):
    sid = lax.axis_index("s")
    cid = lax.axis_index("c")
    wid = sid * NC + cid
    base = pl.multiple_of(wid * BPW, BPW)

    semu = [semu0, semu1, semu2, semu3, semu4, semu5, semu6, semu7]
    semy = [semy0, semy1, semy2, semy3, semy4, semy5, semy6, semy7]

    dio = lax.iota(jnp.int32, L)
    zi = jnp.zeros((L,), jnp.int32)
    zf = jnp.zeros((L,), jnp.float32)

    cpi_u = pltpu.async_copy(user_idx_hbm.at[pl.ds(base, BPW)], uidx_v, semi)
    cpi_i = pltpu.async_copy(item_idx_hbm.at[pl.ds(base, BPW)], iidx_v, semi)
    iu_cps = []
    for srow in range(12):
        iu_cps.append(pltpu.async_copy(
            iu_hbm.at[pl.ds(srow * L, L)], iu1_v.at[pl.ds(srow * L, L)],
            semi2))
    iu_cps.append(pltpu.async_copy(
        iu_hbm.at[pl.ds(184, L)], iu1_v.at[pl.ds(12 * L, L)], semi2))
    cpi_u.wait()
    cpi_i.wait()

    def fire(table, r, buf, slot, sem):
        rt = pl.multiple_of((r // 128) * 128, 128)
        pltpu.async_copy(table.at[:, pl.ds(rt, 128)], buf.at[slot], sem)

    def drain(table, r, buf, slot, sem):
        pltpu.make_async_copy(table.at[:, pl.ds(0, 128)],
                              buf.at[slot], sem).wait()

    rv_u0 = plsc.load_gather(uidx_v, [dio])
    rv_i0 = plsc.load_gather(iidx_v, [dio])
    for k in range(NB):
        fire(ueT_hbm, rv_u0[k], ubuf, k, semu[k])
        fire(ieT_hbm, rv_i0[k], ibuf, k, semu[k])

    cb_u = pltpu.async_copy(ubT_hbm.at[0].at[uidx_v], ub_v, semb)
    cb_i = pltpu.async_copy(ibT_hbm.at[0].at[iidx_v], ib_v, semb)

    for cp in iu_cps:
        cp.wait()

    jidx = jnp.minimum(zi + sid * YJN + dio, 255)
    rv_j = plsc.load_gather(iu1_v, [jidx])
    jr = [rv_j[t] for t in range(YJN)]
    jvalid = [(sid < 12) if t < 8 else (sid < 13) for t in range(YJN)]
    jr = [jnp.where(v, r, 0) for v, r in zip(jvalid, jr)]

    for t in range(YB):
        fire(yjT_hbm, jr[t], yjbuf, t, semy[t])

    f0 = zf
    f1 = zf
    for t in range(YJN):
        drain(yjT_hbm, jr[t], yjbuf, t % YB, semy[t % YB])
        slot = zi + (t % YB)
        rm = zi + (jr[t] % 128)
        c0 = plsc.load_gather(yjbuf, [slot, dio, rm])
        c1 = plsc.load_gather(yjbuf, [slot, dio + L, rm])
        m = (zi == 0) & jvalid[t]
        f0 = f0 + jnp.where(m, c0, 0.0)
        f1 = f1 + jnp.where(m, c1, 0.0)
        if t + YB < YJN:
            fire(yjT_hbm, jr[t + YB], yjbuf, (t + YB) % YB, semy[(t + YB) % YB])

    tmp_v[pl.ds(0, L)] = f0
    tmp_v[pl.ds(L, L)] = f1
    sbase = pl.multiple_of(sid * D, D)
    pltpu.sync_copy(tmp_v, impl_sh.at[pl.ds(sbase, D)])
    plsc.subcore_barrier()
    pltpu.sync_copy(impl_sh, gath_v)

    scale = jnp.float32(1.0 / math.sqrt(HIST))
    f0 = zf
    f1 = zf
    for p in range(NS):
        f0 = f0 + gath_v[pl.ds(p * D, L)]
        f1 = f1 + gath_v[pl.ds(p * D + L, L)]
    f0 = f0 * scale
    f1 = f1 * scale

    cb_u.wait()
    cb_i.wait()

    def chunk_body(chunk, carry):
        cb = chunk * L
        rv_u = plsc.load_gather(uidx_v, [cb + dio])
        rv_i = plsc.load_gather(iidx_v, [cb + dio])
        nxt = jnp.minimum(cb + L + dio, BPW - 1)
        rv_un = plsc.load_gather(uidx_v, [nxt])
        rv_in = plsc.load_gather(iidx_v, [nxt])
        acc = zf
        for kk in range(L):
            slot = kk % NB
            drain(ueT_hbm, rv_u[kk], ubuf, slot, semu[slot])
            drain(ieT_hbm, rv_i[kk], ibuf, slot, semu[slot])
            rm_u = zi + (rv_u[kk] % 128)
            rm_i = zi + (rv_i[kk] % 128)
            sv = zi + slot
            cu0 = plsc.load_gather(ubuf, [sv, dio, rm_u])
            cu1 = plsc.load_gather(ubuf, [sv, dio + L, rm_u])
            ci0 = plsc.load_gather(ibuf, [sv, dio, rm_i])
            ci1 = plsc.load_gather(ibuf, [sv, dio + L, rm_i])
            s = jnp.sum((cu0 + f0) * ci0 + (cu1 + f1) * ci1)
            acc = jnp.where(dio == kk, s, acc)
            kkf = kk + NB
            ru = rv_u[kkf] if kkf < L else rv_un[kkf - L]
            ri = rv_i[kkf] if kkf < L else rv_in[kkf - L]

            @pl.when(cb + kkf < BPW)
            def _(ru=ru, ri=ri, slot=slot):
                fire(ueT_hbm, ru, ubuf, slot, semu[slot])
                fire(ieT_hbm, ri, ibuf, slot, semu[slot])

        ub16 = plsc.load_gather(ub_v, [cb + dio])
        ib16 = plsc.load_gather(ib_v, [cb + dio])
        res = acc + jnp.float32(GLOBAL_MEAN) + ub16 + ib16
        plsc.store_scatter(out_v, [cb + dio], res)
        return carry

    lax.fori_loop(0, NCH, chunk_body, 0)

    pltpu.sync_copy(out_v, out_hbm.at[pl.ds(base, BPW)])


def kernel(user_idx, item_idx, Iu, user_embedding, item_embedding,
           user_bias, item_bias, yj):
    mesh = plsc.VectorSubcoreMesh(core_axis_name="c", subcore_axis_name="s")
    f = pl.kernel(
        _svdpp_kernel,
        mesh=mesh,
        out_type=jax.ShapeDtypeStruct((BATCH,), jnp.float32),
        scratch_types=[
            pltpu.VMEM((BPW,), jnp.int32),
            pltpu.VMEM((BPW,), jnp.int32),
            pltpu.VMEM((256,), jnp.int32),
            pltpu.VMEM((NB, D, 128), jnp.float32),
            pltpu.VMEM((NB, D, 128), jnp.float32),
            pltpu.VMEM((YB, D, 128), jnp.float32),
            pltpu.VMEM((BPW,), jnp.float32),
            pltpu.VMEM((BPW,), jnp.float32),
            pltpu.VMEM((D,), jnp.float32),
            pltpu.VMEM((NS * D,), jnp.float32),
            pltpu.VMEM((BPW,), jnp.float32),
            pltpu.VMEM_SHARED((NS * D,), jnp.float32),
            pltpu.SemaphoreType.DMA,
            pltpu.SemaphoreType.DMA,
            pltpu.SemaphoreType.DMA,
            pltpu.SemaphoreType.DMA,
            pltpu.SemaphoreType.DMA,
            pltpu.SemaphoreType.DMA,
            pltpu.SemaphoreType.DMA,
            pltpu.SemaphoreType.DMA,
            pltpu.SemaphoreType.DMA,
            pltpu.SemaphoreType.DMA,
            pltpu.SemaphoreType.DMA,
            pltpu.SemaphoreType.DMA,
            pltpu.SemaphoreType.DMA,
            pltpu.SemaphoreType.DMA,
            pltpu.SemaphoreType.DMA,
            pltpu.SemaphoreType.DMA,
            pltpu.SemaphoreType.DMA,
            pltpu.SemaphoreType.DMA,
            pltpu.SemaphoreType.DMA,
        ],
        compiler_params=pltpu.CompilerParams(
            needs_layout_passes=False, use_tc_tiling_on_sc=True,
            disable_bounds_checks=True, disable_semaphore_checks=True),
    )
    return f(user_idx, item_idx, Iu, user_embedding.T, item_embedding.T,
             user_bias.T, item_bias.T, yj.T)


# speedup vs baseline: 1.0198x; 1.0198x over previous
import math

import jax
import jax.numpy as jnp
from jax import lax
from jax.experimental import pallas as pl
from jax.experimental.pallas import tpu as pltpu
from jax.experimental.pallas import tpu_sc as plsc

BATCH = 4096
HIST = 200
D = 32
L = 16
GLOBAL_MEAN = 3.5

NC, NS = 2, 16
NW = NC * NS
BPW = BATCH // NW
NCH = BPW // L
NB = 8
YJN = 16
YB = 8
WIDTHS = (32, 64, 96, 128)


def _svdpp_kernel(user_idx_hbm, item_idx_hbm, iu_hbm, ueT_hbm, ieT_hbm,
                  ubT_hbm, ibT_hbm, yjT_hbm, out_hbm,
                  uidx_v, iidx_v, iu1_v, ubuf, ibuf, yjbuf, ub_v, ib_v,
                  tmp_v, gath_v, out_v, it_sav, impl_sh,
                  semu0, semu1, semu2, semu3, semu4, semu5, semu6, semu7,
                  semy0, semy1, semy2, semy3, semy4, semy5, semy6, semy7,
                  semb, semi, semi2):
    sid = lax.axis_index("s")
    cid = lax.axis_index("c")
    wid = sid * NC + cid
    base = pl.multiple_of(wid * BPW, BPW)

    semu = [semu0, semu1, semu2, semu3, semu4, semu5, semu6, semu7]
    semy = [semy0, semy1, semy2, semy3, semy4, semy5, semy6, semy7]

    dio = lax.iota(jnp.int32, L)
    zi = jnp.zeros((L,), jnp.int32)
    zf = jnp.zeros((L,), jnp.float32)

    cpi_u = pltpu.async_copy(user_idx_hbm.at[pl.ds(base, BPW)], uidx_v, semi)
    cpi_i = pltpu.async_copy(item_idx_hbm.at[pl.ds(base, BPW)], iidx_v, semi)
    iu_cps = []
    for srow in range(12):
        iu_cps.append(pltpu.async_copy(
            iu_hbm.at[pl.ds(srow * L, L)], iu1_v.at[pl.ds(srow * L, L)],
            semi2))
    iu_cps.append(pltpu.async_copy(
        iu_hbm.at[pl.ds(184, L)], iu1_v.at[pl.ds(12 * L, L)], semi2))
    cpi_u.wait()
    cpi_i.wait()

    def fire(table, r, buf, slot, sem):
        rt = pl.multiple_of((r // 128) * 128, 128)
        pltpu.async_copy(table.at[:, pl.ds(rt, 128)], buf.at[slot], sem)

    def drain(table, r, buf, slot, sem):
        pltpu.make_async_copy(table.at[:, pl.ds(0, 128)],
                              buf.at[slot], sem).wait()

    rv_u0 = plsc.load_gather(uidx_v, [dio])
    rv_i0 = plsc.load_gather(iidx_v, [dio])
    for k in range(NB):
        fire(ueT_hbm, rv_u0[k], ubuf, k, semu[k])
        fire(ieT_hbm, rv_i0[k], ibuf, k, semu[k])

    cb_u = pltpu.async_copy(ubT_hbm.at[0].at[uidx_v], ub_v, semb)
    cb_i = pltpu.async_copy(ibT_hbm.at[0].at[iidx_v], ib_v, semb)

    for cp in iu_cps:
        cp.wait()

    jidx = jnp.minimum(zi + sid * YJN + dio, 255)
    rv_j = plsc.load_gather(iu1_v, [jidx])
    jr = [rv_j[t] for t in range(YJN)]
    jvalid = [(sid < 12) if t < 8 else (sid < 13) for t in range(YJN)]
    jr = [jnp.where(v, r, 0) for v, r in zip(jvalid, jr)]

    for t in range(YB):
        fire(yjT_hbm, jr[t], yjbuf, t, semy[t])

    cb_u.wait()
    cb_i.wait()

    def chunk_body(chunk, carry):
        cb = chunk * L
        rv_u = plsc.load_gather(uidx_v, [cb + dio])
        rv_i = plsc.load_gather(iidx_v, [cb + dio])
        nxt = jnp.minimum(cb + L + dio, BPW - 1)
        rv_un = plsc.load_gather(uidx_v, [nxt])
        rv_in = plsc.load_gather(iidx_v, [nxt])
        acc = zf
        for kk in range(L):
            slot = kk % NB
            drain(ueT_hbm, rv_u[kk], ubuf, slot, semu[slot])
            drain(ieT_hbm, rv_i[kk], ibuf, slot, semu[slot])
            rm_u = zi + (rv_u[kk] % 128)
            rm_i = zi + (rv_i[kk] % 128)
            sv = zi + slot
            cu0 = plsc.load_gather(ubuf, [sv, dio, rm_u])
            cu1 = plsc.load_gather(ubuf, [sv, dio + L, rm_u])
            ci0 = plsc.load_gather(ibuf, [sv, dio, rm_i])
            ci1 = plsc.load_gather(ibuf, [sv, dio + L, rm_i])
            col = zi + cb + kk
            plsc.store_scatter(it_sav, [dio, col], ci0)
            plsc.store_scatter(it_sav, [dio + L, col], ci1)
            s = jnp.sum(cu0 * ci0 + cu1 * ci1)
            acc = jnp.where(dio == kk, s, acc)
            kkf = kk + NB
            ru = rv_u[kkf] if kkf < L else rv_un[kkf - L]
            ri = rv_i[kkf] if kkf < L else rv_in[kkf - L]

            @pl.when(cb + kkf < BPW)
            def _(ru=ru, ri=ri, slot=slot):
                fire(ueT_hbm, ru, ubuf, slot, semu[slot])
                fire(ieT_hbm, ri, ibuf, slot, semu[slot])

        ub16 = plsc.load_gather(ub_v, [cb + dio])
        ib16 = plsc.load_gather(ib_v, [cb + dio])
        res = acc + jnp.float32(GLOBAL_MEAN) + ub16 + ib16
        plsc.store_scatter(out_v, [cb + dio], res)
        return carry

    lax.fori_loop(0, NCH, chunk_body, 0)

    f0 = zf
    f1 = zf
    for t in range(YJN):
        drain(yjT_hbm, jr[t], yjbuf, t % YB, semy[t % YB])
        slot = zi + (t % YB)
        rm = zi + (jr[t] % 128)
        c0 = plsc.load_gather(yjbuf, [slot, dio, rm])
        c1 = plsc.load_gather(yjbuf, [slot, dio + L, rm])
        m = (zi == 0) & jvalid[t]
        f0 = f0 + jnp.where(m, c0, 0.0)
        f1 = f1 + jnp.where(m, c1, 0.0)
        if t + YB < YJN:
            fire(yjT_hbm, jr[t + YB], yjbuf, (t + YB) % YB, semy[(t + YB) % YB])

    tmp_v[pl.ds(0, L)] = f0
    tmp_v[pl.ds(L, L)] = f1
    sbase = pl.multiple_of(sid * D, D)
    pltpu.sync_copy(tmp_v, impl_sh.at[pl.ds(sbase, D)])
    plsc.subcore_barrier()
    pltpu.sync_copy(impl_sh, gath_v)

    scale = jnp.float32(1.0 / math.sqrt(HIST))
    f0 = zf
    f1 = zf
    for p in range(NS):
        f0 = f0 + gath_v[pl.ds(p * D, L)]
        f1 = f1 + gath_v[pl.ds(p * D + L, L)]
    f0 = f0 * scale
    f1 = f1 * scale

    fs = [f0[d] for d in range(L)] + [f1[d] for d in range(L)]

    def impl_body(chunk, carry):
        cb = chunk * L
        cols = cb + dio
        acc2 = plsc.load_gather(out_v, [cols])
        for d in range(D):
            acc2 = acc2 + fs[d] * plsc.load_gather(it_sav, [zi + d, cols])
        plsc.store_scatter(out_v, [cols], acc2)
        return carry

    lax.fori_loop(0, NCH, impl_body, 0)

    pltpu.sync_copy(out_v, out_hbm.at[pl.ds(base, BPW)])


def kernel(user_idx, item_idx, Iu, user_embedding, item_embedding,
           user_bias, item_bias, yj):
    mesh = plsc.VectorSubcoreMesh(core_axis_name="c", subcore_axis_name="s")
    f = pl.kernel(
        _svdpp_kernel,
        mesh=mesh,
        out_type=jax.ShapeDtypeStruct((BATCH,), jnp.float32),
        scratch_types=[
            pltpu.VMEM((BPW,), jnp.int32),
            pltpu.VMEM((BPW,), jnp.int32),
            pltpu.VMEM((256,), jnp.int32),
            pltpu.VMEM((NB, D, 128), jnp.float32),
            pltpu.VMEM((NB, D, 128), jnp.float32),
            pltpu.VMEM((YB, D, 128), jnp.float32),
            pltpu.VMEM((BPW,), jnp.float32),
            pltpu.VMEM((BPW,), jnp.float32),
            pltpu.VMEM((D,), jnp.float32),
            pltpu.VMEM((NS * D,), jnp.float32),
            pltpu.VMEM((BPW,), jnp.float32),
            pltpu.VMEM((D, BPW), jnp.float32),
            pltpu.VMEM_SHARED((NS * D,), jnp.float32),
            pltpu.SemaphoreType.DMA,
            pltpu.SemaphoreType.DMA,
            pltpu.SemaphoreType.DMA,
            pltpu.SemaphoreType.DMA,
            pltpu.SemaphoreType.DMA,
            pltpu.SemaphoreType.DMA,
            pltpu.SemaphoreType.DMA,
            pltpu.SemaphoreType.DMA,
            pltpu.SemaphoreType.DMA,
            pltpu.SemaphoreType.DMA,
            pltpu.SemaphoreType.DMA,
            pltpu.SemaphoreType.DMA,
            pltpu.SemaphoreType.DMA,
            pltpu.SemaphoreType.DMA,
            pltpu.SemaphoreType.DMA,
            pltpu.SemaphoreType.DMA,
            pltpu.SemaphoreType.DMA,
            pltpu.SemaphoreType.DMA,
            pltpu.SemaphoreType.DMA,
        ],
        compiler_params=pltpu.CompilerParams(
            needs_layout_passes=False, use_tc_tiling_on_sc=True),
    )
    return f(user_idx, item_idx, Iu, user_embedding.T, item_embedding.T,
             user_bias.T, item_bias.T, yj.T)
